# baseline (speedup 1.0000x reference)
import functools

import jax
import jax.numpy as jnp
from jax import lax
from jax.experimental import pallas as pl
from jax.experimental.pallas import tpu as pltpu
from jax.experimental.pallas import tpu_sc as plsc

D = 32
L = 16
JB = 8
IB = 128
NTOK = JB * IB


def _build_sc_kernel(b, xlen, units_per_worker):
    info = plsc.get_sparse_core_info()
    nc, ns = info.num_cores, info.num_subcores
    n_jh = xlen // JB
    n_ib = b // IB
    mesh = plsc.VectorSubcoreMesh(core_axis_name="c", subcore_axis_name="s")

    @functools.partial(
        pl.kernel,
        out_type=jax.ShapeDtypeStruct((xlen, D // 8, n_ib, 8, IB),
                                      jnp.float32),
        mesh=mesh,
        compiler_params=pltpu.CompilerParams(
            use_tc_tiling_on_sc=False, needs_layout_passes=False),
        scratch_types=[
            pltpu.VMEM((2, JB, IB), jnp.int32),
            pltpu.VMEM((2, NTOK, D), jnp.float32),
            pltpu.VMEM((JB, D, L), jnp.float32),
            pltpu.VMEM((JB, D // 8, 8, IB), jnp.float32),
            pltpu.SemaphoreType.DMA,
            pltpu.SemaphoreType.DMA,
            pltpu.SemaphoreType.DMA,
        ],
    )
    def k(x4d_hbm, tok_hbm, posx_hbm, out_hbm,
          idx_v, tokr_v, posm_v, outb_v, semg0, semg1, semo):
        wid = lax.axis_index("s") * nc + lax.axis_index("c")
        u0 = wid * units_per_worker
        semg = (semg0, semg1)

        def fire(uid, bb):
            jh = uid // n_ib
            ib = uid % n_ib
            pltpu.sync_copy(x4d_hbm.at[jh, ib], idx_v.at[bb])
            for g in range(JB):
                pltpu.async_copy(
                    tok_hbm.at[idx_v.at[bb, g]],
                    tokr_v.at[bb, pl.ds(g * IB, IB)], semg[bb])

        iota = lax.iota(jnp.int32, L)

        def compute(uid, bb, first):
            jh = uid // n_ib
            ib = uid % n_ib
            for g in range(JB):
                pltpu.make_async_copy(
                    tok_hbm.at[idx_v.at[bb, g]],
                    tokr_v.at[bb, pl.ds(g * IB, IB)], semg[bb]).wait()
            pltpu.sync_copy(posx_hbm.at[pl.ds(jh * JB + 1, JB)], posm_v)

            @pl.when(jnp.logical_not(first))
            def _():
                def dr(t, c2):
                    pltpu.make_async_copy(
                        outb_v.at[0, 0], out_hbm.at[0, 0, 0], semo).wait()
                    return c2
                lax.fori_loop(0, JB * (D // 8), dr, 0)

            def col_body(jl, c2):
                rbase = jl * IB
                for g in range(IB // L):
                    row0 = rbase + g * L
                    rowvec = row0 + iota
                    iv = idx_v[bb, jl, pl.ds(g * L, L)]
                    msk = iv > 0
                    for dh in range(D // 8):
                        for dl in range(8):
                            d = dh * 8 + dl
                            vals = plsc.load_gather(
                                tokr_v.at[bb],
                                [rowvec, jnp.full((L,), d, jnp.int32)])
                            posd = posm_v[jl, d, :]
                            psel = jnp.where(msk, posd,
                                             jnp.zeros((L,), jnp.float32))
                            outb_v[jl, dh, dl, pl.ds(g * L, L)] = vals + psel
                return c2
            lax.fori_loop(0, JB, col_body, 0)

            def sc_body(jl, c2):
                for dh in range(D // 8):
                    pltpu.async_copy(
                        outb_v.at[jl, dh],
                        out_hbm.at[jh * JB + jl, dh, ib], semo)
                return c2
            lax.fori_loop(0, JB, sc_body, 0)

        fire(u0, 0)
        npairs = units_per_worker // 2

        def pair_body(p, carry):
            u = u0 + p * 2
            fire(u + 1, 1)
            compute(u, 0, p == 0)
            if units_per_worker % 2 == 1:
                fire(u + 2, 0)
            else:
                @pl.when(p + 1 < npairs)
                def _():
                    fire(u + 2, 0)
            compute(u + 1, 1, jnp.bool_(False))
            return carry
        lax.fori_loop(0, npairs, pair_body, 0)
        if units_per_worker % 2 == 1:
            compute(u0 + units_per_worker - 1, 0, jnp.bool_(False))

        def drf(t, c2):
            pltpu.make_async_copy(
                outb_v.at[0, 0], out_hbm.at[0, 0, 0], semo).wait()
            return c2
        lax.fori_loop(0, JB * (D // 8), drf, 0)

    return k


@jax.jit
def kernel(x, token_table, pos_table):
    b, xlen = x.shape
    info = plsc.get_sparse_core_info()
    nw = info.num_cores * info.num_subcores
    n_units = (xlen // JB) * (b // IB)
    units_per_worker = n_units // nw

    xi = x.astype(jnp.int32)
    x4d = xi.reshape(b // IB, IB, xlen // JB, JB).transpose(2, 0, 3, 1)
    posx = jnp.broadcast_to(pos_table[:, :, None], (xlen + 1, D, L))

    out5d = _build_sc_kernel(b, xlen, units_per_worker)(
        x4d, token_table, posx)
    return out5d.transpose(2, 4, 0, 1, 3).reshape(b, xlen, D)
